# baseline (speedup 1.0000x reference)
import functools

import jax
import jax.numpy as jnp
from jax import lax
from jax.experimental import pallas as pl
from jax.experimental.pallas import tpu as pltpu

_K = 16


@jax.jit
def _gather_rows_tc(idx, table):
    (N,) = idx.shape
    V, D = table.shape

    def body(idx_ref, table_ref, out_ref, sem):
        def issue(i):
            pltpu.make_async_copy(
                table_ref.at[pl.ds(idx_ref[i], 1), :],
                out_ref.at[pl.ds(i, 1), :],
                sem,
            ).start()

        def wait_one(i):
            pltpu.make_async_copy(
                table_ref.at[pl.ds(0, 1), :],
                out_ref.at[pl.ds(i, 1), :],
                sem,
            ).wait()

        for i in range(_K):
            issue(i)

        def step(i, carry):
            wait_one(i - _K)
            issue(i)
            return carry

        lax.fori_loop(_K, N, step, 0)

        def drain(i, carry):
            wait_one(i)
            return carry

        lax.fori_loop(N - _K, N, drain, 0)

    grid_spec = pltpu.PrefetchScalarGridSpec(
        num_scalar_prefetch=1,
        grid=(1,),
        in_specs=[pl.BlockSpec(memory_space=pltpu.MemorySpace.HBM)],
        out_specs=pl.BlockSpec(memory_space=pltpu.MemorySpace.HBM),
        scratch_shapes=[pltpu.SemaphoreType.DMA],
    )
    return pl.pallas_call(
        body,
        grid_spec=grid_spec,
        out_shape=jax.ShapeDtypeStruct((N, D), jnp.float32),
    )(idx, table)


def kernel(X, table):
    B, T = X.shape
    idx = X.reshape(B * T).astype(jnp.int32)
    out = _gather_rows_tc(idx, table)
    return out.reshape(B, T, table.shape[1])


# speedup vs baseline: 6.8550x; 6.8550x over previous
import functools

import jax
import jax.numpy as jnp
from jax import lax
from jax.experimental import pallas as pl
from jax.experimental.pallas import tpu as pltpu

_R = 8


@jax.jit
def _gather_rows_tc(idx, table3):
    (N,) = idx.shape
    V, _, D = table3.shape

    def body(idx_ref, *refs):
        n = _R
        ins = refs[:n]
        out = refs[n]
        for r in range(n):
            out[r, 0, :] = ins[r][0, 0, :]

    grid_spec = pltpu.PrefetchScalarGridSpec(
        num_scalar_prefetch=1,
        grid=(N // _R,),
        in_specs=[
            pl.BlockSpec((1, 1, D), functools.partial(
                lambda r, i, idx_ref: (idx_ref[i * _R + r], 0, 0), r))
            for r in range(_R)
        ],
        out_specs=pl.BlockSpec((_R, 1, D), lambda i, idx_ref: (i, 0, 0)),
    )
    return pl.pallas_call(
        body,
        grid_spec=grid_spec,
        out_shape=jax.ShapeDtypeStruct((N, 1, D), jnp.float32),
    )(idx, *([table3] * _R))


def kernel(X, table):
    B, T = X.shape
    V, D = table.shape
    idx = X.reshape(B * T).astype(jnp.int32)
    out = _gather_rows_tc(idx, table.reshape(V, 1, D))
    return out.reshape(B, T, D)


# speedup vs baseline: 39.3459x; 5.7397x over previous
import functools

import jax
import jax.numpy as jnp
from jax import lax
from jax.experimental import pallas as pl
from jax.experimental.pallas import tpu as pltpu
from jax.experimental.pallas import tpu_sc as plsc

_INFO = plsc.get_sparse_core_info()
_NC, _NS = _INFO.num_cores, _INFO.num_subcores
_NW = _NC * _NS

_C = 4
_NBUF = 2


@jax.jit
def _gather_rows(idx2, table):
    n_rows_total, c = idx2.shape
    N = n_rows_total * c
    V, D = table.shape
    b_per_w = N // _NW
    n_chunks = b_per_w // _C
    n_steady = n_chunks // _NBUF - 1
    mesh = plsc.VectorSubcoreMesh(core_axis_name="c", subcore_axis_name="s")

    @functools.partial(
        pl.kernel,
        mesh=mesh,
        out_type=jax.ShapeDtypeStruct((N, D), jnp.float32),
        scratch_types=[
            pltpu.VMEM((n_chunks, _C), jnp.int32),
            pltpu.VMEM((_NBUF, _C, D), jnp.float32),
            pltpu.SemaphoreType.DMA,
            pltpu.SemaphoreType.DMA,
            pltpu.SemaphoreType.DMA,
            pltpu.SemaphoreType.DMA,
        ],
    )
    def body(idx_hbm, table_hbm, out_hbm, idx_v, rows_v, g0, g1, s0, s1):
        gsem = (g0, g1)
        ssem = (s0, s1)
        wid = lax.axis_index("s") * _NC + lax.axis_index("c")
        base = wid * b_per_w
        pltpu.sync_copy(idx_hbm.at[pl.ds(wid * n_chunks, n_chunks), :], idx_v)

        def gather(ch, b):
            pltpu.async_copy(table_hbm.at[idx_v.at[ch]], rows_v.at[b], gsem[b])

        def scatter(ch, b):
            pltpu.async_copy(
                rows_v.at[b], out_hbm.at[pl.ds(base + ch * _C, _C)], ssem[b])

        for b in range(_NBUF):
            gather(b, b)

        def step(s, carry):
            for b in range(_NBUF):
                ch = s * _NBUF + b
                pltpu.make_async_copy(
                    table_hbm.at[idx_v.at[ch]], rows_v.at[b], gsem[b]).wait()
                scatter(ch, b)
                pltpu.make_async_copy(
                    rows_v.at[b], out_hbm.at[pl.ds(base + ch * _C, _C)],
                    ssem[b]).wait()
                gather(ch + _NBUF, b)
            return carry

        lax.fori_loop(0, n_steady, step, 0)

        for b in range(_NBUF):
            ch = n_chunks - _NBUF + b
            pltpu.make_async_copy(
                table_hbm.at[idx_v.at[ch]], rows_v.at[b], gsem[b]).wait()
            scatter(ch, b)
        for b in range(_NBUF):
            ch = n_chunks - _NBUF + b
            pltpu.make_async_copy(
                rows_v.at[b], out_hbm.at[pl.ds(base + ch * _C, _C)],
                ssem[b]).wait()

    return body(idx2, table)


def kernel(X, table):
    B, T = X.shape
    idx2 = X.reshape(B * T // _C, _C).astype(jnp.int32)
    out = _gather_rows(idx2, table)
    return out.reshape(B, T, table.shape[1])
